# baseline (speedup 1.0000x reference)
import functools

import jax
import jax.numpy as jnp
from jax import lax
from jax.experimental import pallas as pl
from jax.experimental.pallas import tpu as pltpu
from jax.experimental.pallas import tpu_sc as plsc

_EPS = 1e-8


def _inv_norm(x):
    s = jnp.sum(x * x, axis=1, keepdims=True)
    return jnp.where(s > _EPS * _EPS, lax.rsqrt(s), 1.0 / _EPS)


def _sc_gather(table, pos_pairs, neg_pairs):
    V, D = table.shape
    P = pos_pairs.shape[1]
    K = neg_pairs.shape[1]
    info = plsc.get_sparse_core_info()
    NW = info.num_cores * info.num_subcores
    chunk = P // NW
    assert chunk == 128 and K == P
    mesh = plsc.VectorSubcoreMesh(core_axis_name="c", subcore_axis_name="s")

    half = chunk // 2

    @functools.partial(
        pl.kernel,
        mesh=mesh,
        out_type=jax.ShapeDtypeStruct((2 * P + K, D), jnp.float32),
        scratch_types=[
            pltpu.VMEM((6, half), jnp.int32),
            pltpu.VMEM((6, half, D), jnp.float32),
            pltpu.SemaphoreType.DMA,
            pltpu.SemaphoreType.DMA,
        ],
    )
    def gather_kernel(table_hbm, pp_hbm, np_hbm, out_hbm, idx_v, rows_v,
                      gsem, wsem):
        wid = lax.axis_index("s") * info.num_cores + lax.axis_index("c")
        off = wid * chunk
        srcs = [(pp_hbm, 0), (pp_hbm, 1), (np_hbm, 1)]
        for j in range(3):
            src, row = srcs[j]
            pltpu.sync_copy(src.at[row, pl.ds(off, half)], idx_v.at[2 * j])
            pltpu.sync_copy(src.at[row, pl.ds(off + half, half)],
                            idx_v.at[2 * j + 1])
        gathers = [
            pltpu.async_copy(table_hbm.at[idx_v.at[c]], rows_v.at[c], gsem)
            for c in range(6)
        ]
        writes = []
        for c in range(6):
            gathers[c].wait()
            j, h = c // 2, c % 2
            writes.append(
                pltpu.async_copy(
                    rows_v.at[c],
                    out_hbm.at[pl.ds(j * P + off + h * half, half)],
                    wsem))
        for w in writes:
            w.wait()

    return gather_kernel(table, pos_pairs, neg_pairs)


def _flash_body(P, NB, KB, n_valid, temp_ref, a_ref, pos_ref, neg_ref,
                lg_ref, lb_ref, out_ref, an_ref, ps_ref, acc_ref):
    k = pl.program_id(0)
    inv_t = 1.0 / temp_ref[0]
    log2e = 1.4426950408889634

    @pl.when(k == 0)
    def _init():
        a = a_ref[...]
        a_n = a * _inv_norm(a)
        an_ref[...] = (a_n * (inv_t * log2e)).astype(jnp.bfloat16)
        p = pos_ref[...]
        p_n = p * _inv_norm(p)
        ps2 = jnp.sum(a_n * p_n, axis=1, keepdims=True) * (inv_t * log2e)
        ps_ref[...] = ps2
        acc_ref[...] = jnp.zeros_like(acc_ref)
        acc_ref[:, :1] = jnp.exp2(ps2)

    nb = neg_ref[...]
    n_n = nb * _inv_norm(nb)
    sims2 = lax.dot_general(
        an_ref[...], n_n.astype(jnp.bfloat16), (((1,), (1,)), ((), ())),
        preferred_element_type=jnp.float32,
        precision=lax.Precision.DEFAULT)
    e = jnp.exp2(sims2.astype(jnp.bfloat16))
    part = sum(e[:, i * 128:(i + 1) * 128] for i in range(NB // 128))
    acc_ref[...] += part.astype(jnp.float32)

    @pl.when(k == KB - 1)
    def _finish():
        row = jnp.sum(acc_ref[...], axis=1, keepdims=True)
        per_anchor = (jnp.log2(row) - ps_ref[...]) / log2e
        nce = jnp.sum(per_anchor) / P
        lg = lg_ref[...]
        lb = lb_ref[...]
        sp = jnp.maximum(-lg, 0.0) + jnp.log1p(jnp.exp(-jnp.abs(lg)))
        bce = jnp.sum(sp + (1.0 - lb) * lg) / n_valid
        out_ref[0, 0] = 0.5 * bce + nce


def _flash_loss(temperature, gathered, logits_pad, labels_pad, P, K, F,
                n_valid):
    NB = 2048
    assert K % NB == 0
    KB = K // NB
    rows_l, lanes = logits_pad.shape
    body = functools.partial(_flash_body, P, NB, KB, n_valid)
    out = pl.pallas_call(
        body,
        grid=(KB,),
        in_specs=[
            pl.BlockSpec(memory_space=pltpu.SMEM),
            pl.BlockSpec((P, F), lambda k: (0, 0)),
            pl.BlockSpec((P, F), lambda k: (1, 0)),
            pl.BlockSpec((NB, F), lambda k: (2 * P // NB + k, 0)),
            pl.BlockSpec((rows_l, lanes), lambda k: (0, 0)),
            pl.BlockSpec((rows_l, lanes), lambda k: (0, 0)),
        ],
        out_specs=pl.BlockSpec(memory_space=pltpu.SMEM),
        out_shape=jax.ShapeDtypeStruct((1, 1), jnp.float32),
        scratch_shapes=[
            pltpu.VMEM((P, F), jnp.bfloat16),
            pltpu.VMEM((P, 1), jnp.float32),
            pltpu.VMEM((P, 128), jnp.float32),
        ],
    )(jnp.reshape(temperature, (1,)), gathered, gathered, gathered,
      logits_pad, labels_pad)
    return out[0, 0]


def kernel(logits, labels, node_embeddings, pos_pairs, neg_pairs, temperature):
    N, F = node_embeddings.shape
    P = pos_pairs.shape[1]
    K = neg_pairs.shape[1]

    gathered = _sc_gather(node_embeddings, pos_pairs, neg_pairs)

    lg = jnp.reshape(jnp.squeeze(logits), (-1,))
    n_valid = lg.shape[0]
    cols = 125 if n_valid % 125 == 0 else 128
    assert n_valid % cols == 0
    lg_pad = jnp.reshape(lg, (-1, cols))
    lb_pad = jnp.reshape(labels, (-1, cols))

    return _flash_loss(temperature.astype(jnp.float32), gathered,
                       lg_pad, lb_pad, P, K, F, n_valid)


# speedup vs baseline: 1.0427x; 1.0427x over previous
import functools

import jax
import jax.numpy as jnp
from jax import lax
from jax.experimental import pallas as pl
from jax.experimental.pallas import tpu as pltpu
from jax.experimental.pallas import tpu_sc as plsc

_EPS = 1e-8


def _inv_norm(x):
    s = jnp.sum(x * x, axis=1, keepdims=True)
    return jnp.where(s > _EPS * _EPS, lax.rsqrt(s), 1.0 / _EPS)


def _sc_gather(table, pos_pairs, neg_pairs):
    V, D = table.shape
    P = pos_pairs.shape[1]
    K = neg_pairs.shape[1]
    info = plsc.get_sparse_core_info()
    NW = info.num_cores * info.num_subcores
    chunk = P // NW
    assert chunk == 128 and K == P
    mesh = plsc.VectorSubcoreMesh(core_axis_name="c", subcore_axis_name="s")

    @functools.partial(
        pl.kernel,
        mesh=mesh,
        out_type=jax.ShapeDtypeStruct((2 * P + K, D), jnp.float32),
        scratch_types=[
            pltpu.VMEM((3, chunk), jnp.int32),
            pltpu.VMEM((3, chunk, D), jnp.float32),
            pltpu.SemaphoreType.DMA,
            pltpu.SemaphoreType.DMA,
        ],
    )
    def gather_kernel(table_hbm, pp_hbm, np_hbm, out_hbm, idx_v, rows_v,
                      gsem, wsem):
        wid = lax.axis_index("s") * info.num_cores + lax.axis_index("c")
        off = wid * chunk
        pltpu.sync_copy(pp_hbm.at[0, pl.ds(off, chunk)], idx_v.at[0])
        pltpu.sync_copy(pp_hbm.at[1, pl.ds(off, chunk)], idx_v.at[1])
        pltpu.sync_copy(np_hbm.at[1, pl.ds(off, chunk)], idx_v.at[2])
        gathers = [
            pltpu.async_copy(table_hbm.at[idx_v.at[j]], rows_v.at[j], gsem)
            for j in range(3)
        ]
        writes = []
        for j in range(3):
            gathers[j].wait()
            writes.append(
                pltpu.async_copy(rows_v.at[j],
                                 out_hbm.at[pl.ds(j * P + off, chunk)],
                                 wsem))
        for w in writes:
            w.wait()

    return gather_kernel(table, pos_pairs, neg_pairs)


def _flash_body(P, NB, KB, n_valid, temp_ref, a_ref, pos_ref, neg_ref,
                lg_ref, lb_ref, out_ref, an_ref, ps_ref, acc_ref):
    k = pl.program_id(0)
    inv_t = 1.0 / temp_ref[0]
    log2e = 1.4426950408889634

    @pl.when(k == 0)
    def _init():
        a = a_ref[...]
        a_n = a * _inv_norm(a)
        an_ref[...] = (a_n * (inv_t * log2e)).astype(jnp.bfloat16)
        p = pos_ref[...]
        p_n = p * _inv_norm(p)
        ps2 = jnp.sum(a_n * p_n, axis=1, keepdims=True) * (inv_t * log2e)
        ps_ref[...] = ps2
        acc_ref[...] = jnp.zeros_like(acc_ref)
        acc_ref[:, :1] = jnp.exp2(ps2)

    nb = neg_ref[...]
    n_n = nb * _inv_norm(nb)
    sims2 = lax.dot_general(
        an_ref[...], n_n.astype(jnp.bfloat16), (((1,), (1,)), ((), ())),
        preferred_element_type=jnp.float32,
        precision=lax.Precision.DEFAULT)
    e = jnp.exp2(sims2.astype(jnp.bfloat16))
    part = sum(e[:, i * 128:(i + 1) * 128] for i in range(NB // 128))
    acc_ref[...] += part.astype(jnp.float32)

    @pl.when(k == KB - 1)
    def _finish():
        row = jnp.sum(acc_ref[...], axis=1, keepdims=True)
        per_anchor = (jnp.log2(row) - ps_ref[...]) / log2e
        nce = jnp.sum(per_anchor) / P
        lg = lg_ref[...]
        lb = lb_ref[...]
        sp = jnp.maximum(-lg, 0.0) + jnp.log1p(jnp.exp(-jnp.abs(lg)))
        bce = jnp.sum(sp + (1.0 - lb) * lg) / n_valid
        out_ref[0, 0] = 0.5 * bce + nce


def _flash_loss(temperature, gathered, logits_pad, labels_pad, P, K, F,
                n_valid):
    NB = 2048
    assert K % NB == 0
    KB = K // NB
    rows_l, lanes = logits_pad.shape
    body = functools.partial(_flash_body, P, NB, KB, n_valid)
    out = pl.pallas_call(
        body,
        grid=(KB,),
        in_specs=[
            pl.BlockSpec(memory_space=pltpu.SMEM),
            pl.BlockSpec((P, F), lambda k: (0, 0)),
            pl.BlockSpec((P, F), lambda k: (1, 0)),
            pl.BlockSpec((NB, F), lambda k: (2 * P // NB + k, 0)),
            pl.BlockSpec((rows_l, lanes), lambda k: (0, 0)),
            pl.BlockSpec((rows_l, lanes), lambda k: (0, 0)),
        ],
        out_specs=pl.BlockSpec(memory_space=pltpu.SMEM),
        out_shape=jax.ShapeDtypeStruct((1, 1), jnp.float32),
        scratch_shapes=[
            pltpu.VMEM((P, F), jnp.bfloat16),
            pltpu.VMEM((P, 1), jnp.float32),
            pltpu.VMEM((P, 128), jnp.float32),
        ],
    )(jnp.reshape(temperature, (1,)), gathered, gathered, gathered,
      logits_pad, labels_pad)
    return out[0, 0]


def kernel(logits, labels, node_embeddings, pos_pairs, neg_pairs, temperature):
    N, F = node_embeddings.shape
    P = pos_pairs.shape[1]
    K = neg_pairs.shape[1]

    gathered = _sc_gather(node_embeddings, pos_pairs, neg_pairs)

    lg = jnp.reshape(jnp.squeeze(logits), (-1,))
    n_valid = lg.shape[0]
    cols = 125 if n_valid % 125 == 0 else 128
    assert n_valid % cols == 0
    lg_pad = jnp.reshape(lg, (-1, cols))
    lb_pad = jnp.reshape(labels, (-1, cols))

    return _flash_loss(temperature.astype(jnp.float32), gathered,
                       lg_pad, lb_pad, P, K, F, n_valid)
